# baseline (speedup 1.0000x reference)
import functools

import jax
import jax.numpy as jnp
import numpy as np
from jax import lax
from jax.experimental import pallas as pl
from jax.experimental.pallas import tpu as pltpu
from jax.experimental.pallas import tpu_sc as plsc

N_VOXELS = 40000
N_POINTS = 160000
N_GRID = 32

PT_BLK = 2048
NP_PAD = 163840
N_ROWS = NP_PAD // 128
N_BLOCKS = NP_PAD // PT_BLK
NV_PAD = 40960
VOX_CHUNK = 512

NC, NS, L = 2, 16, 16
NW = NC * NS
T0_ROWS = NV_PAD
TK_ROWS = 131072
CHUNK = 256
PPW = NP_PAD // NW
PPT = NP_PAD // NS
G_STEPS = PPW // CHUNK
S_STEPS = PPT // CHUNK

_INTERPRET = False

_CRANGE = np.array([[-51.2, 51.2], [-51.2, 51.2], [-4.0, 2.4]], dtype=np.float32)
_SPATIAL = np.array([512.0, 512.0, 32.0], dtype=np.float32)
_INTERVALS = (_CRANGE[:, 1] - _CRANGE[:, 0]) / _SPATIAL
_MINS = _CRANGE[:, 0]


def _sc_body(xs_h, ys_h, zs_h, vd_h, bi_h, x0_h, x1_h, y0_h, y1_h, z0_h,
             z1_h, ci_h, out_h, rec_h, *refs):
    tabs = refs[:16]
    (st_i, st_f, idx0, idx1, idx2, idx3, chanv, obuf, rbuf,
     zbuf, sem) = refs[16:]
    c = lax.axis_index("c")
    s = lax.axis_index("s")
    w = c * NS + s

    def zf(i, _):
        zbuf[pl.ds(i * L, L)] = jnp.zeros((L,), jnp.float32)
        return _
    lax.fori_loop(0, 1024 // L, zf, 0)

    ZC = 1024
    t0_pt = T0_ROWS // NS
    for t in tabs[:4]:
        for k in range(t0_pt // ZC):
            pltpu.sync_copy(zbuf, t.at[pl.ds(s * t0_pt + k * ZC, ZC)])
        pltpu.sync_copy(zbuf.at[pl.ds(0, t0_pt % ZC)],
                        t.at[pl.ds(s * t0_pt + (t0_pt // ZC) * ZC,
                                   t0_pt % ZC)])
    tk_pt = TK_ROWS // NS
    for t in tabs[4:]:
        for k in range(tk_pt // ZC):
            pltpu.sync_copy(zbuf, t.at[pl.ds(s * tk_pt + k * ZC, ZC)])
    plsc.subcore_barrier()

    CR = CHUNK // 128

    def _stage(r0):
        ds_ = [
            pltpu.async_copy(bi_h.at[pl.ds(r0, CR), :], st_i.at[0], sem),
            pltpu.async_copy(x0_h.at[pl.ds(r0, CR), :], st_i.at[1], sem),
            pltpu.async_copy(x1_h.at[pl.ds(r0, CR), :], st_i.at[2], sem),
            pltpu.async_copy(y0_h.at[pl.ds(r0, CR), :], st_i.at[3], sem),
            pltpu.async_copy(y1_h.at[pl.ds(r0, CR), :], st_i.at[4], sem),
            pltpu.async_copy(z0_h.at[pl.ds(r0, CR), :], st_i.at[5], sem),
            pltpu.async_copy(z1_h.at[pl.ds(r0, CR), :], st_i.at[6], sem),
            pltpu.async_copy(ci_h.at[pl.ds(r0, CR), :], idx0, sem),
            pltpu.async_copy(xs_h.at[pl.ds(r0, CR), :], st_f.at[0], sem),
            pltpu.async_copy(ys_h.at[pl.ds(r0, CR), :], st_f.at[1], sem),
            pltpu.async_copy(zs_h.at[pl.ds(r0, CR), :], st_f.at[2], sem),
            pltpu.async_copy(vd_h.at[pl.ds(r0, CR), :], st_f.at[3], sem),
        ]
        for d in ds_:
            d.wait()

    def _idxcompute(k):
        def body(sl, _):
            o = sl * L
            bi = st_i[0, k, pl.ds(o, L)]
            x0 = st_i[1, k, pl.ds(o, L)]
            x1 = st_i[2, k, pl.ds(o, L)]
            y0 = st_i[3, k, pl.ds(o, L)]
            y1 = st_i[4, k, pl.ds(o, L)]
            z0 = st_i[5, k, pl.ds(o, L)]
            z1 = st_i[6, k, pl.ds(o, L)]
            idx1[k, pl.ds(o, L)] = ((bi * N_GRID + x1) * N_GRID + y0) * N_GRID + z0
            idx2[k, pl.ds(o, L)] = ((bi * N_GRID + x0) * N_GRID + y1) * N_GRID + z0
            idx3[k, pl.ds(o, L)] = ((bi * N_GRID + x0) * N_GRID + y0) * N_GRID + z1
            return _
        lax.fori_loop(0, 128 // L, body, 0)

    def _sphase(g, carry):
        _stage(s * (PPT // 128) + g * (CHUNK // 128))
        ds_ = []
        for k in range(CHUNK // 128):
            _idxcompute(k)
            for idx, base in ((idx0, 0), (idx1, 4), (idx2, 8), (idx3, 12)):
                ir = idx.at[k]
                for ch in range(4):
                    ds_.append(pltpu.async_copy(st_f.at[ch, k],
                                                tabs[base + ch].at[ir], sem,
                                                add=True))
        for d in ds_:
            d.wait()
        return carry
    lax.fori_loop(0, S_STEPS, _sphase, 0)
    plsc.subcore_barrier()

    def _gphase(g, carry):
        r0 = w * (PPW // 128) + g * (CHUNK // 128)
        _stage(r0)
        ds_ = []
        for k in range(CHUNK // 128):
            _idxcompute(k)
            for idx, base in ((idx0, 0), (idx1, 4), (idx2, 8), (idx3, 12)):
                ir = idx.at[k]
                for ch in range(4):
                    ds_.append(pltpu.async_copy(tabs[base + ch].at[ir],
                                                chanv.at[base + ch, k], sem))
        for d in ds_:
            d.wait()
        for k in range(CHUNK // 128):
            def gbody(sl, _):
                o = sl * L
                xs = st_f[0, k, pl.ds(o, L)]
                ys = st_f[1, k, pl.ds(o, L)]
                zs = st_f[2, k, pl.ds(o, L)]
                n0 = jnp.maximum(chanv[3, k, pl.ds(o, L)], 1.0)
                obuf[0, k, pl.ds(o, L)] = chanv[0, k, pl.ds(o, L)] / n0
                obuf[1, k, pl.ds(o, L)] = chanv[1, k, pl.ds(o, L)] / n0
                obuf[2, k, pl.ds(o, L)] = chanv[2, k, pl.ds(o, L)] / n0
                rbuf[k, pl.ds(o, L)] = 1.0 / n0

                def ak(base, xi, yi, zi, cb):
                    nk = jnp.maximum(chanv[base + 3, k, pl.ds(o, L)], 1.0)
                    mx = chanv[base + 0, k, pl.ds(o, L)] / nk
                    my = chanv[base + 1, k, pl.ds(o, L)] / nk
                    mz = chanv[base + 2, k, pl.ds(o, L)] / nk
                    dx = xs - (xi.astype(jnp.float32) * _INTERVALS[0] + _MINS[0])
                    dy = ys - (yi.astype(jnp.float32) * _INTERVALS[1] + _MINS[1])
                    dz = zs - (zi.astype(jnp.float32) * _INTERVALS[2] + _MINS[2])
                    n2 = dx * dx + dy * dy + dz * dz
                    bits = lax.bitcast_convert_type(n2, jnp.int32)
                    y = lax.bitcast_convert_type(
                        jnp.int32(0x5F3759DF) - lax.shift_right_logical(bits, 1),
                        jnp.float32)
                    for _i in range(3):
                        y = y * (1.5 - 0.5 * n2 * y * y)
                    obuf[cb + 0, k, pl.ds(o, L)] = (xs - mx) * y
                    obuf[cb + 1, k, pl.ds(o, L)] = (ys - my) * y
                    obuf[cb + 2, k, pl.ds(o, L)] = (zs - mz) * y

                x0 = st_i[1, k, pl.ds(o, L)]
                x1 = st_i[2, k, pl.ds(o, L)]
                y0 = st_i[3, k, pl.ds(o, L)]
                y1 = st_i[4, k, pl.ds(o, L)]
                z0 = st_i[5, k, pl.ds(o, L)]
                z1 = st_i[6, k, pl.ds(o, L)]
                ak(4, x1, y0, z0, 3)
                ak(8, x0, y1, z0, 6)
                ak(12, x0, y0, z1, 9)
                return _
            lax.fori_loop(0, 128 // L, gbody, 0)

        ds_ = [pltpu.async_copy(obuf.at[ch], out_h.at[ch, pl.ds(r0, CR), :],
                                sem) for ch in range(12)]
        ds_.append(pltpu.async_copy(rbuf, rec_h.at[pl.ds(r0, CR), :], sem))
        for d in ds_:
            d.wait()
        return carry
    lax.fori_loop(0, G_STEPS, _gphase, 0)


def _sc_features(xs, ys, zs, vd, bi, x0, x1, y0, y1, z0, z1, ci):
    mesh = plsc.VectorSubcoreMesh(core_axis_name="c", subcore_axis_name="s",
                                  num_cores=NC, num_subcores=NS)
    f = pl.kernel(
        _sc_body,
        out_type=[jax.ShapeDtypeStruct((12, N_ROWS, 128), jnp.float32),
                  jax.ShapeDtypeStruct((N_ROWS, 128), jnp.float32)],
        mesh=mesh,
        scratch_types=(
            [pltpu.VMEM_SHARED((T0_ROWS,), jnp.float32)] * 4 +
            [pltpu.VMEM_SHARED((TK_ROWS,), jnp.float32)] * 12 +
            [
                pltpu.VMEM((7, CHUNK // 128, 128), jnp.int32),
                pltpu.VMEM((4, CHUNK // 128, 128), jnp.float32),
                pltpu.VMEM((CHUNK // 128, 128), jnp.int32),
                pltpu.VMEM((CHUNK // 128, 128), jnp.int32),
                pltpu.VMEM((CHUNK // 128, 128), jnp.int32),
                pltpu.VMEM((CHUNK // 128, 128), jnp.int32),
                pltpu.VMEM((16, CHUNK // 128, 128), jnp.float32),
                pltpu.VMEM((12, CHUNK // 128, 128), jnp.float32),
                pltpu.VMEM((CHUNK // 128, 128), jnp.float32),
                pltpu.VMEM((1024,), jnp.float32),
                pltpu.SemaphoreType.DMA,
            ]),
    )
    return f(xs, ys, zs, vd, bi, x0, x1, y0, y1, z0, z1, ci)


def _tc_body(coors_ref, pts_ref, fc_ref, sc_ref, recip_ref, w1p_ref, w1g_ref,
             w1s_ref, b1_ref, w2_ref, b2_ref, out_ref):
    i = pl.program_id(0)

    @pl.when(i == 0)
    def _init():
        out_ref[...] = jnp.zeros_like(out_ref)

    h = jax.lax.dot_general(pts_ref[...], w1p_ref[...], (((1,), (0,)), ((), ())),
                            preferred_element_type=jnp.float32)
    h += jax.lax.dot_general(fc_ref[...].astype(jnp.float32), w1g_ref[...],
                             (((1,), (0,)), ((), ())),
                             preferred_element_type=jnp.float32)
    h += jax.lax.dot_general(sc_ref[...], w1s_ref[...], (((0,), (0,)), ((), ())),
                             preferred_element_type=jnp.float32)
    h = jnp.maximum(h + b1_ref[...], 0.0)
    pf = jax.lax.dot_general(h, w2_ref[...], (((1,), (0,)), ((), ())),
                             preferred_element_type=jnp.float32)
    pf = (pf + b2_ref[...]) * recip_ref[...]

    idx = coors_ref[0, 0, :]
    lo = coors_ref[0, 0, 0]
    hi = coors_ref[0, 0, PT_BLK - 1]
    lo_al = (lo // VOX_CHUNK) * VOX_CHUNK
    nchunk = (hi - lo_al) // VOX_CHUNK + 1
    idx_row = idx.reshape(1, PT_BLK)

    def chunk_body(ch, carry):
        vbase = lo_al + ch * VOX_CHUNK
        rows = jax.lax.broadcasted_iota(jnp.int32, (VOX_CHUNK, PT_BLK), 0) + vbase
        oh = jnp.where(rows == idx_row, 1.0, 0.0)
        contrib = jax.lax.dot_general(oh, pf, (((1,), (0,)), ((), ())),
                                      preferred_element_type=jnp.float32)
        out_ref[pl.ds(vbase, VOX_CHUNK), :] += contrib
        return carry

    jax.lax.fori_loop(0, nchunk, chunk_body, 0)


def _mlp_segmean(coors3, pts, fc, sc12, recip, W1p, W1g, W1s, b1e, W2, b2r):
    return pl.pallas_call(
        _tc_body,
        grid=(N_BLOCKS,),
        in_specs=[
            pl.BlockSpec((1, 1, PT_BLK), lambda i: (i, 0, 0)),
            pl.BlockSpec((PT_BLK, 4), lambda i: (i, 0)),
            pl.BlockSpec((PT_BLK, 4), lambda i: (i, 0)),
            pl.BlockSpec((12, PT_BLK), lambda i: (0, i)),
            pl.BlockSpec((PT_BLK, 1), lambda i: (i, 0)),
            pl.BlockSpec((4, 64), lambda i: (0, 0)),
            pl.BlockSpec((4, 64), lambda i: (0, 0)),
            pl.BlockSpec((12, 64), lambda i: (0, 0)),
            pl.BlockSpec((1, 64), lambda i: (0, 0)),
            pl.BlockSpec((64, 64), lambda i: (0, 0)),
            pl.BlockSpec((1, 64), lambda i: (0, 0)),
        ],
        out_specs=pl.BlockSpec((NV_PAD, 64), lambda i: (0, 0)),
        out_shape=jax.ShapeDtypeStruct((NV_PAD, 64), jnp.float32),
        interpret=_INTERPRET,
    )(coors3, pts, fc, sc12, recip, W1p, W1g, W1s, b1e, W2, b2r)


def _pad1(a, padval):
    pad = NP_PAD - N_POINTS
    return jnp.concatenate([a, jnp.full((pad,), padval, a.dtype)])


def kernel(points, batch_idx, full_coors, coors_inv, xidx0, xidx1, yidx0,
           yidx1, zidx0, zidx1, W1, bias1, W2, bias2):
    pad = NP_PAD - N_POINTS
    ar = jnp.arange(pad, dtype=jnp.int32)

    xs = _pad1(points[:, 0], 0.0).reshape(N_ROWS, 128)
    ys = _pad1(points[:, 1], 0.0).reshape(N_ROWS, 128)
    zs = _pad1(points[:, 2], 0.0).reshape(N_ROWS, 128)
    vd = jnp.concatenate([jnp.ones((N_POINTS,), jnp.float32),
                          jnp.zeros((pad,), jnp.float32)]).reshape(N_ROWS, 128)
    bi = _pad1(batch_idx, 0).reshape(N_ROWS, 128)
    x0p = _pad1(xidx0, 0).reshape(N_ROWS, 128)
    x1p = _pad1(xidx1, 0).reshape(N_ROWS, 128)
    y0p = _pad1(yidx0, 0).reshape(N_ROWS, 128)
    y1p = _pad1(yidx1, 0).reshape(N_ROWS, 128)
    z0p = _pad1(zidx0, 0).reshape(N_ROWS, 128)
    z1p = _pad1(zidx1, 0).reshape(N_ROWS, 128)
    cip = jnp.concatenate([coors_inv, N_VOXELS + (ar % (NV_PAD - N_VOXELS))])
    ci2 = cip.reshape(N_ROWS, 128)

    sc12, rec = _sc_features(xs, ys, zs, vd, bi, x0p, x1p, y0p, y1p, z0p, z1p,
                             ci2)
    sc12 = sc12.reshape(12, NP_PAD)
    recip = rec.reshape(NP_PAD, 1)

    iv = _INTERVALS
    mn = _MINS
    W1p = jnp.stack([
        W1[0] + W1[4] + W1[7],
        W1[1] + W1[5] + W1[8],
        W1[2] + W1[6] + W1[9],
        W1[3],
    ])
    W1g = jnp.stack([
        jnp.zeros((64,), jnp.float32),
        -iv[0] * W1[7],
        -iv[1] * W1[8],
        -iv[2] * W1[9],
    ])
    b1e = (bias1 - mn[0] * W1[7] - mn[1] * W1[8] - mn[2] * W1[9]).reshape(1, 64)
    W1s = jnp.concatenate([-W1[4:7], W1[10:19]])

    pts_p = jnp.concatenate([points, jnp.zeros((pad, 4), jnp.float32)], axis=0)
    fc_p = jnp.concatenate([full_coors, jnp.zeros((pad, 4), jnp.int32)],
                           axis=0)
    coors3 = cip.reshape(N_BLOCKS, 1, PT_BLK)

    out = _mlp_segmean(coors3, pts_p, fc_p, sc12, recip, W1p, W1g, W1s, b1e,
                       W2, bias2.reshape(1, 64))
    return out[:N_VOXELS]


# speedup vs baseline: 1.3624x; 1.3624x over previous
import functools

import jax
import jax.numpy as jnp
import numpy as np
from jax import lax
from jax.experimental import pallas as pl
from jax.experimental.pallas import tpu as pltpu
from jax.experimental.pallas import tpu_sc as plsc

N_VOXELS = 40000
N_POINTS = 160000
N_GRID = 32

PT_BLK = 2048
NP_PAD = 163840
N_ROWS = NP_PAD // 128
N_BLOCKS = NP_PAD // PT_BLK
NV_PAD = 40960
VOX_CHUNK = 512

NC, NS, L = 2, 16, 16
NW = NC * NS
T0_ROWS = NV_PAD
TK_ROWS = 131072
CHUNK = 256
PPW = NP_PAD // NW
PPT = NP_PAD // NS
G_STEPS = PPW // CHUNK
S_STEPS = PPT // CHUNK

_INTERPRET = False

_CRANGE = np.array([[-51.2, 51.2], [-51.2, 51.2], [-4.0, 2.4]], dtype=np.float32)
_SPATIAL = np.array([512.0, 512.0, 32.0], dtype=np.float32)
_INTERVALS = (_CRANGE[:, 1] - _CRANGE[:, 0]) / _SPATIAL
_MINS = _CRANGE[:, 0]


def _sc_body(xs_h, ys_h, zs_h, vd_h, bi_h, x0_h, x1_h, y0_h, y1_h, z0_h,
             z1_h, ci_h, out_h, rec_h, *refs):
    tabs = refs[:16]
    (st_i, st_f, idx0, idx1, idx2, idx3, chanv, obuf, rbuf,
     zbuf, sem) = refs[16:]
    c = lax.axis_index("c")
    s = lax.axis_index("s")
    w = c * NS + s

    def zf(i, _):
        zbuf[pl.ds(i * L, L)] = jnp.zeros((L,), jnp.float32)
        return _
    lax.fori_loop(0, 1024 // L, zf, 0)

    ZC = 1024
    t0_pt = T0_ROWS // NS
    for t in tabs[:4]:
        for k in range(t0_pt // ZC):
            pltpu.sync_copy(zbuf, t.at[pl.ds(s * t0_pt + k * ZC, ZC)])
        pltpu.sync_copy(zbuf.at[pl.ds(0, t0_pt % ZC)],
                        t.at[pl.ds(s * t0_pt + (t0_pt // ZC) * ZC,
                                   t0_pt % ZC)])
    tk_pt = TK_ROWS // NS
    for t in tabs[4:]:
        for k in range(tk_pt // ZC):
            pltpu.sync_copy(zbuf, t.at[pl.ds(s * tk_pt + k * ZC, ZC)])
    plsc.subcore_barrier()

    CR = CHUNK // 128

    def _stage(r0):
        ds_ = [
            pltpu.async_copy(bi_h.at[pl.ds(r0, CR), :], st_i.at[0], sem),
            pltpu.async_copy(x0_h.at[pl.ds(r0, CR), :], st_i.at[1], sem),
            pltpu.async_copy(x1_h.at[pl.ds(r0, CR), :], st_i.at[2], sem),
            pltpu.async_copy(y0_h.at[pl.ds(r0, CR), :], st_i.at[3], sem),
            pltpu.async_copy(y1_h.at[pl.ds(r0, CR), :], st_i.at[4], sem),
            pltpu.async_copy(z0_h.at[pl.ds(r0, CR), :], st_i.at[5], sem),
            pltpu.async_copy(z1_h.at[pl.ds(r0, CR), :], st_i.at[6], sem),
            pltpu.async_copy(ci_h.at[pl.ds(r0, CR), :], idx0, sem),
            pltpu.async_copy(xs_h.at[pl.ds(r0, CR), :], st_f.at[0], sem),
            pltpu.async_copy(ys_h.at[pl.ds(r0, CR), :], st_f.at[1], sem),
            pltpu.async_copy(zs_h.at[pl.ds(r0, CR), :], st_f.at[2], sem),
            pltpu.async_copy(vd_h.at[pl.ds(r0, CR), :], st_f.at[3], sem),
        ]
        for d in ds_:
            d.wait()

    def _idxcompute(k):
        def body(sl, _):
            o = sl * L
            bi = st_i[0, k, pl.ds(o, L)]
            x0 = st_i[1, k, pl.ds(o, L)]
            x1 = st_i[2, k, pl.ds(o, L)]
            y0 = st_i[3, k, pl.ds(o, L)]
            y1 = st_i[4, k, pl.ds(o, L)]
            z0 = st_i[5, k, pl.ds(o, L)]
            z1 = st_i[6, k, pl.ds(o, L)]
            idx1[k, pl.ds(o, L)] = ((bi * N_GRID + x1) * N_GRID + y0) * N_GRID + z0
            idx2[k, pl.ds(o, L)] = ((bi * N_GRID + x0) * N_GRID + y1) * N_GRID + z0
            idx3[k, pl.ds(o, L)] = ((bi * N_GRID + x0) * N_GRID + y0) * N_GRID + z1
            return _
        lax.fori_loop(0, 128 // L, body, 0)

    def _sphase(g, carry):
        _stage(s * (PPT // 128) + g * (CHUNK // 128))

        def _scat(groups):
            ds_ = []
            for k in range(CHUNK // 128):
                _idxcompute(k)
                for idx, base in groups:
                    ir = idx.at[k]
                    for ch in range(4):
                        ds_.append(pltpu.async_copy(st_f.at[ch, k],
                                                    tabs[base + ch].at[ir],
                                                    sem, add=True))
            for d in ds_:
                d.wait()

        @pl.when(c == 0)
        def _():
            _scat(((idx0, 0), (idx1, 4)))

        @pl.when(c == 1)
        def _():
            _scat(((idx2, 8), (idx3, 12)))
        return carry
    lax.fori_loop(0, S_STEPS, _sphase, 0)
    plsc.subcore_barrier()

    def _gphase(g, carry):
        r0 = s * (PPT // 128) + g * (CHUNK // 128)
        _stage(r0)

        def _gat(groups):
            ds_ = []
            for k in range(CHUNK // 128):
                _idxcompute(k)
                for idx, base in groups:
                    ir = idx.at[k]
                    for ch in range(4):
                        ds_.append(pltpu.async_copy(tabs[base + ch].at[ir],
                                                    chanv.at[base + ch, k],
                                                    sem))
            for d in ds_:
                d.wait()

        def _ak(k, sl, base, xi, yi, zi, cb):
            o = sl * L
            xs = st_f[0, k, pl.ds(o, L)]
            ys = st_f[1, k, pl.ds(o, L)]
            zs = st_f[2, k, pl.ds(o, L)]
            nk = jnp.maximum(chanv[base + 3, k, pl.ds(o, L)], 1.0)
            mx = chanv[base + 0, k, pl.ds(o, L)] / nk
            my = chanv[base + 1, k, pl.ds(o, L)] / nk
            mz = chanv[base + 2, k, pl.ds(o, L)] / nk
            dx = xs - (xi.astype(jnp.float32) * _INTERVALS[0] + _MINS[0])
            dy = ys - (yi.astype(jnp.float32) * _INTERVALS[1] + _MINS[1])
            dz = zs - (zi.astype(jnp.float32) * _INTERVALS[2] + _MINS[2])
            n2 = dx * dx + dy * dy + dz * dz
            bits = lax.bitcast_convert_type(n2, jnp.int32)
            y = lax.bitcast_convert_type(
                jnp.int32(0x5F3759DF) - lax.shift_right_logical(bits, 1),
                jnp.float32)
            for _i in range(3):
                y = y * (1.5 - 0.5 * n2 * y * y)
            obuf[cb + 0, k, pl.ds(o, L)] = (xs - mx) * y
            obuf[cb + 1, k, pl.ds(o, L)] = (ys - my) * y
            obuf[cb + 2, k, pl.ds(o, L)] = (zs - mz) * y

        @pl.when(c == 0)
        def _():
            _gat(((idx0, 0), (idx1, 4)))
            for k in range(CHUNK // 128):
                def gbody0(sl, _u):
                    o = sl * L
                    n0 = jnp.maximum(chanv[3, k, pl.ds(o, L)], 1.0)
                    obuf[0, k, pl.ds(o, L)] = chanv[0, k, pl.ds(o, L)] / n0
                    obuf[1, k, pl.ds(o, L)] = chanv[1, k, pl.ds(o, L)] / n0
                    obuf[2, k, pl.ds(o, L)] = chanv[2, k, pl.ds(o, L)] / n0
                    rbuf[k, pl.ds(o, L)] = 1.0 / n0
                    _ak(k, sl, 4,
                        st_i[2, k, pl.ds(o, L)], st_i[3, k, pl.ds(o, L)],
                        st_i[5, k, pl.ds(o, L)], 3)
                    return _u
                lax.fori_loop(0, 128 // L, gbody0, 0)
            ds_ = [pltpu.async_copy(obuf.at[ch],
                                    out_h.at[ch, pl.ds(r0, CR), :], sem)
                   for ch in range(6)]
            ds_.append(pltpu.async_copy(rbuf, rec_h.at[pl.ds(r0, CR), :], sem))
            for d in ds_:
                d.wait()

        @pl.when(c == 1)
        def _():
            _gat(((idx2, 8), (idx3, 12)))
            for k in range(CHUNK // 128):
                def gbody1(sl, _u):
                    o = sl * L
                    _ak(k, sl, 8,
                        st_i[1, k, pl.ds(o, L)], st_i[4, k, pl.ds(o, L)],
                        st_i[5, k, pl.ds(o, L)], 6)
                    _ak(k, sl, 12,
                        st_i[1, k, pl.ds(o, L)], st_i[3, k, pl.ds(o, L)],
                        st_i[6, k, pl.ds(o, L)], 9)
                    return _u
                lax.fori_loop(0, 128 // L, gbody1, 0)
            ds_ = [pltpu.async_copy(obuf.at[ch],
                                    out_h.at[ch, pl.ds(r0, CR), :], sem)
                   for ch in range(6, 12)]
            for d in ds_:
                d.wait()
        return carry
    lax.fori_loop(0, S_STEPS, _gphase, 0)


def _sc_features(xs, ys, zs, vd, bi, x0, x1, y0, y1, z0, z1, ci):
    mesh = plsc.VectorSubcoreMesh(core_axis_name="c", subcore_axis_name="s",
                                  num_cores=NC, num_subcores=NS)
    f = pl.kernel(
        _sc_body,
        out_type=[jax.ShapeDtypeStruct((12, N_ROWS, 128), jnp.float32),
                  jax.ShapeDtypeStruct((N_ROWS, 128), jnp.float32)],
        mesh=mesh,
        scratch_types=(
            [pltpu.VMEM_SHARED((T0_ROWS,), jnp.float32)] * 4 +
            [pltpu.VMEM_SHARED((TK_ROWS,), jnp.float32)] * 12 +
            [
                pltpu.VMEM((7, CHUNK // 128, 128), jnp.int32),
                pltpu.VMEM((4, CHUNK // 128, 128), jnp.float32),
                pltpu.VMEM((CHUNK // 128, 128), jnp.int32),
                pltpu.VMEM((CHUNK // 128, 128), jnp.int32),
                pltpu.VMEM((CHUNK // 128, 128), jnp.int32),
                pltpu.VMEM((CHUNK // 128, 128), jnp.int32),
                pltpu.VMEM((16, CHUNK // 128, 128), jnp.float32),
                pltpu.VMEM((12, CHUNK // 128, 128), jnp.float32),
                pltpu.VMEM((CHUNK // 128, 128), jnp.float32),
                pltpu.VMEM((1024,), jnp.float32),
                pltpu.SemaphoreType.DMA,
            ]),
    )
    return f(xs, ys, zs, vd, bi, x0, x1, y0, y1, z0, z1, ci)


def _tc_body(coors_ref, p8_ref, sc_ref, recip_ref, w1e_ref, w1s_ref, b1_ref,
             w2_ref, b2_ref, out_ref):
    i = pl.program_id(0)

    @pl.when(i == 0)
    def _init():
        out_ref[...] = jnp.zeros_like(out_ref)

    h = jax.lax.dot_general(p8_ref[...], w1e_ref[...], (((1,), (0,)), ((), ())),
                            preferred_element_type=jnp.float32)
    h += jax.lax.dot_general(sc_ref[...], w1s_ref[...], (((0,), (0,)), ((), ())),
                             preferred_element_type=jnp.float32)
    h = jnp.maximum(h + b1_ref[...], 0.0)
    pf = jax.lax.dot_general(h, w2_ref[...], (((1,), (0,)), ((), ())),
                             preferred_element_type=jnp.float32)
    pf = (pf + b2_ref[...]) * recip_ref[...]

    idx = coors_ref[0, 0, :]
    lo = coors_ref[0, 0, 0]
    hi = coors_ref[0, 0, PT_BLK - 1]
    lo_al = (lo // VOX_CHUNK) * VOX_CHUNK
    nchunk = (hi - lo_al) // VOX_CHUNK + 1
    idx_row = idx.reshape(1, PT_BLK)

    def chunk_body(ch, carry):
        vbase = lo_al + ch * VOX_CHUNK
        rows = jax.lax.broadcasted_iota(jnp.int32, (VOX_CHUNK, PT_BLK), 0) + vbase
        oh = jnp.where(rows == idx_row, 1.0, 0.0)
        contrib = jax.lax.dot_general(oh, pf, (((1,), (0,)), ((), ())),
                                      preferred_element_type=jnp.float32)
        out_ref[pl.ds(vbase, VOX_CHUNK), :] += contrib
        return carry

    jax.lax.fori_loop(0, nchunk, chunk_body, 0)


def _mlp_segmean(coors3, p8, sc12, recip, W1e, W1s, b1e, W2, b2r):
    return pl.pallas_call(
        _tc_body,
        grid=(N_BLOCKS,),
        in_specs=[
            pl.BlockSpec((1, 1, PT_BLK), lambda i: (i, 0, 0)),
            pl.BlockSpec((PT_BLK, 8), lambda i: (i, 0)),
            pl.BlockSpec((12, PT_BLK), lambda i: (0, i)),
            pl.BlockSpec((PT_BLK, 1), lambda i: (i, 0)),
            pl.BlockSpec((8, 64), lambda i: (0, 0)),
            pl.BlockSpec((12, 64), lambda i: (0, 0)),
            pl.BlockSpec((1, 64), lambda i: (0, 0)),
            pl.BlockSpec((64, 64), lambda i: (0, 0)),
            pl.BlockSpec((1, 64), lambda i: (0, 0)),
        ],
        out_specs=pl.BlockSpec((NV_PAD, 64), lambda i: (0, 0)),
        out_shape=jax.ShapeDtypeStruct((NV_PAD, 64), jnp.float32),
        interpret=_INTERPRET,
    )(coors3, p8, sc12, recip, W1e, W1s, b1e, W2, b2r)


def _pad1(a, padval):
    pad = NP_PAD - N_POINTS
    return jnp.concatenate([a, jnp.full((pad,), padval, a.dtype)])


def kernel(points, batch_idx, full_coors, coors_inv, xidx0, xidx1, yidx0,
           yidx1, zidx0, zidx1, W1, bias1, W2, bias2):
    pad = NP_PAD - N_POINTS
    ar = jnp.arange(pad, dtype=jnp.int32)

    xs = _pad1(points[:, 0], 0.0).reshape(N_ROWS, 128)
    ys = _pad1(points[:, 1], 0.0).reshape(N_ROWS, 128)
    zs = _pad1(points[:, 2], 0.0).reshape(N_ROWS, 128)
    vd = jnp.concatenate([jnp.ones((N_POINTS,), jnp.float32),
                          jnp.zeros((pad,), jnp.float32)]).reshape(N_ROWS, 128)
    bi = _pad1(batch_idx, 0).reshape(N_ROWS, 128)
    x0p = _pad1(xidx0, 0).reshape(N_ROWS, 128)
    x1p = _pad1(xidx1, 0).reshape(N_ROWS, 128)
    y0p = _pad1(yidx0, 0).reshape(N_ROWS, 128)
    y1p = _pad1(yidx1, 0).reshape(N_ROWS, 128)
    z0p = _pad1(zidx0, 0).reshape(N_ROWS, 128)
    z1p = _pad1(zidx1, 0).reshape(N_ROWS, 128)
    cip = jnp.concatenate([coors_inv, N_VOXELS + (ar % (NV_PAD - N_VOXELS))])
    ci2 = cip.reshape(N_ROWS, 128)

    sc12, rec = _sc_features(xs, ys, zs, vd, bi, x0p, x1p, y0p, y1p, z0p, z1p,
                             ci2)
    sc12 = sc12.reshape(12, NP_PAD)
    recip = rec.reshape(NP_PAD, 1)

    iv = _INTERVALS
    mn = _MINS
    W1e = jnp.stack([
        W1[0] + W1[4] + W1[7],
        W1[1] + W1[5] + W1[8],
        W1[2] + W1[6] + W1[9],
        W1[3],
        -iv[0] * W1[7],
        -iv[1] * W1[8],
        -iv[2] * W1[9],
        jnp.zeros((64,), jnp.float32),
    ])
    b1e = (bias1 - mn[0] * W1[7] - mn[1] * W1[8] - mn[2] * W1[9]).reshape(1, 64)
    W1s = jnp.concatenate([-W1[4:7], W1[10:19]])

    p8 = jnp.concatenate([
        points, full_coors[:, 1:].astype(jnp.float32),
        jnp.zeros((N_POINTS, 1), jnp.float32)], axis=1)
    p8 = jnp.concatenate([p8, jnp.zeros((pad, 8), jnp.float32)], axis=0)
    coors3 = cip.reshape(N_BLOCKS, 1, PT_BLK)

    out = _mlp_segmean(coors3, p8, sc12, recip, W1e, W1s, b1e, W2,
                       bias2.reshape(1, 64))
    return out[:N_VOXELS]


# speedup vs baseline: 1.3636x; 1.0009x over previous
import functools

import jax
import jax.numpy as jnp
import numpy as np
from jax import lax
from jax.experimental import pallas as pl
from jax.experimental.pallas import tpu as pltpu
from jax.experimental.pallas import tpu_sc as plsc

N_VOXELS = 40000
N_POINTS = 160000
N_GRID = 32

PT_BLK = 2048
NP_PAD = 163840
N_ROWS = NP_PAD // 128
N_BLOCKS = NP_PAD // PT_BLK
NV_PAD = 40960
VOX_CHUNK = 512

NC, NS, L = 2, 16, 16
NW = NC * NS
T0_ROWS = NV_PAD
TK_ROWS = 131072
CHUNK = 256
PPW = NP_PAD // NW
PPT = NP_PAD // NS
G_STEPS = PPW // CHUNK
S_STEPS = PPT // CHUNK

_INTERPRET = False

_CRANGE = np.array([[-51.2, 51.2], [-51.2, 51.2], [-4.0, 2.4]], dtype=np.float32)
_SPATIAL = np.array([512.0, 512.0, 32.0], dtype=np.float32)
_INTERVALS = (_CRANGE[:, 1] - _CRANGE[:, 0]) / _SPATIAL
_MINS = _CRANGE[:, 0]


def _sc_body(xs_h, ys_h, zs_h, vd_h, bi_h, x0_h, x1_h, y0_h, y1_h, z0_h,
             z1_h, ci_h, out_h, rec_h, *refs):
    tabs = refs[:16]
    (st_i, st_f, idx0, idx1, idx2, idx3, chanv, obuf, rbuf,
     zbuf, sem) = refs[16:]
    c = lax.axis_index("c")
    s = lax.axis_index("s")
    w = c * NS + s

    def zf(i, _):
        zbuf[pl.ds(i * L, L)] = jnp.zeros((L,), jnp.float32)
        return _
    lax.fori_loop(0, 1024 // L, zf, 0)

    ZC = 1024
    t0_pt = T0_ROWS // NS
    for t in tabs[:4]:
        for k in range(t0_pt // ZC):
            pltpu.sync_copy(zbuf, t.at[pl.ds(s * t0_pt + k * ZC, ZC)])
        pltpu.sync_copy(zbuf.at[pl.ds(0, t0_pt % ZC)],
                        t.at[pl.ds(s * t0_pt + (t0_pt // ZC) * ZC,
                                   t0_pt % ZC)])
    tk_pt = TK_ROWS // NS
    for t in tabs[4:]:
        for k in range(tk_pt // ZC):
            pltpu.sync_copy(zbuf, t.at[pl.ds(s * tk_pt + k * ZC, ZC)])
    plsc.subcore_barrier()

    CR = CHUNK // 128

    def _stage(r0):
        ds_ = [
            pltpu.async_copy(bi_h.at[pl.ds(r0, CR), :], st_i.at[0], sem),
            pltpu.async_copy(x0_h.at[pl.ds(r0, CR), :], st_i.at[1], sem),
            pltpu.async_copy(x1_h.at[pl.ds(r0, CR), :], st_i.at[2], sem),
            pltpu.async_copy(y0_h.at[pl.ds(r0, CR), :], st_i.at[3], sem),
            pltpu.async_copy(y1_h.at[pl.ds(r0, CR), :], st_i.at[4], sem),
            pltpu.async_copy(z0_h.at[pl.ds(r0, CR), :], st_i.at[5], sem),
            pltpu.async_copy(z1_h.at[pl.ds(r0, CR), :], st_i.at[6], sem),
            pltpu.async_copy(ci_h.at[pl.ds(r0, CR), :], idx0, sem),
            pltpu.async_copy(xs_h.at[pl.ds(r0, CR), :], st_f.at[0], sem),
            pltpu.async_copy(ys_h.at[pl.ds(r0, CR), :], st_f.at[1], sem),
            pltpu.async_copy(zs_h.at[pl.ds(r0, CR), :], st_f.at[2], sem),
            pltpu.async_copy(vd_h.at[pl.ds(r0, CR), :], st_f.at[3], sem),
        ]
        for d in ds_:
            d.wait()

    def _idxcompute(k):
        def body(sl, _):
            o = sl * L
            bi = st_i[0, k, pl.ds(o, L)]
            x0 = st_i[1, k, pl.ds(o, L)]
            x1 = st_i[2, k, pl.ds(o, L)]
            y0 = st_i[3, k, pl.ds(o, L)]
            y1 = st_i[4, k, pl.ds(o, L)]
            z0 = st_i[5, k, pl.ds(o, L)]
            z1 = st_i[6, k, pl.ds(o, L)]
            idx1[k, pl.ds(o, L)] = ((bi * N_GRID + x1) * N_GRID + y0) * N_GRID + z0
            idx2[k, pl.ds(o, L)] = ((bi * N_GRID + x0) * N_GRID + y1) * N_GRID + z0
            idx3[k, pl.ds(o, L)] = ((bi * N_GRID + x0) * N_GRID + y0) * N_GRID + z1
            return _
        lax.fori_loop(0, 128 // L, body, 0)

    def _sphase(g, carry):
        _stage(s * (PPT // 128) + g * (CHUNK // 128))

        def _scat(groups):
            ds_ = []
            for k in range(CHUNK // 128):
                _idxcompute(k)
                for idx, base in groups:
                    ir = idx.at[k]
                    for ch in range(4):
                        ds_.append(pltpu.async_copy(st_f.at[ch, k],
                                                    tabs[base + ch].at[ir],
                                                    sem, add=True))
            for d in ds_:
                d.wait()

        @pl.when(c == 0)
        def _():
            _scat(((idx0, 0), (idx1, 4)))

        @pl.when(c == 1)
        def _():
            _scat(((idx2, 8), (idx3, 12)))
        return carry
    lax.fori_loop(0, S_STEPS, _sphase, 0)
    plsc.subcore_barrier()

    def _gphase(g, carry):
        r0 = s * (PPT // 128) + g * (CHUNK // 128)
        _stage(r0)

        def _gat(groups):
            ds_ = []
            for k in range(CHUNK // 128):
                _idxcompute(k)
                for idx, base in groups:
                    ir = idx.at[k]
                    for ch in range(4):
                        ds_.append(pltpu.async_copy(tabs[base + ch].at[ir],
                                                    chanv.at[base + ch, k],
                                                    sem))
            for d in ds_:
                d.wait()

        def _ak(k, sl, base, xi, yi, zi, cb):
            o = sl * L
            xs = st_f[0, k, pl.ds(o, L)]
            ys = st_f[1, k, pl.ds(o, L)]
            zs = st_f[2, k, pl.ds(o, L)]
            nk = jnp.maximum(chanv[base + 3, k, pl.ds(o, L)], 1.0)
            mx = chanv[base + 0, k, pl.ds(o, L)] / nk
            my = chanv[base + 1, k, pl.ds(o, L)] / nk
            mz = chanv[base + 2, k, pl.ds(o, L)] / nk
            dx = xs - (xi.astype(jnp.float32) * _INTERVALS[0] + _MINS[0])
            dy = ys - (yi.astype(jnp.float32) * _INTERVALS[1] + _MINS[1])
            dz = zs - (zi.astype(jnp.float32) * _INTERVALS[2] + _MINS[2])
            n2 = dx * dx + dy * dy + dz * dz
            bits = lax.bitcast_convert_type(n2, jnp.int32)
            y = lax.bitcast_convert_type(
                jnp.int32(0x5F3759DF) - lax.shift_right_logical(bits, 1),
                jnp.float32)
            for _i in range(3):
                y = y * (1.5 - 0.5 * n2 * y * y)
            obuf[cb + 0, k, pl.ds(o, L)] = (xs - mx) * y
            obuf[cb + 1, k, pl.ds(o, L)] = (ys - my) * y
            obuf[cb + 2, k, pl.ds(o, L)] = (zs - mz) * y

        @pl.when(c == 0)
        def _():
            _gat(((idx0, 0), (idx1, 4)))
            for k in range(CHUNK // 128):
                def gbody0(sl, _u):
                    o = sl * L
                    n0 = jnp.maximum(chanv[3, k, pl.ds(o, L)], 1.0)
                    obuf[0, k, pl.ds(o, L)] = chanv[0, k, pl.ds(o, L)] / n0
                    obuf[1, k, pl.ds(o, L)] = chanv[1, k, pl.ds(o, L)] / n0
                    obuf[2, k, pl.ds(o, L)] = chanv[2, k, pl.ds(o, L)] / n0
                    rbuf[k, pl.ds(o, L)] = 1.0 / n0
                    _ak(k, sl, 4,
                        st_i[2, k, pl.ds(o, L)], st_i[3, k, pl.ds(o, L)],
                        st_i[5, k, pl.ds(o, L)], 3)
                    return _u
                lax.fori_loop(0, 128 // L, gbody0, 0)
            ds_ = [pltpu.async_copy(obuf.at[ch],
                                    out_h.at[ch, pl.ds(r0, CR), :], sem)
                   for ch in range(6)]
            ds_.append(pltpu.async_copy(rbuf, rec_h.at[pl.ds(r0, CR), :], sem))
            for d in ds_:
                d.wait()

        @pl.when(c == 1)
        def _():
            _gat(((idx2, 8), (idx3, 12)))
            for k in range(CHUNK // 128):
                def gbody1(sl, _u):
                    o = sl * L
                    _ak(k, sl, 8,
                        st_i[1, k, pl.ds(o, L)], st_i[4, k, pl.ds(o, L)],
                        st_i[5, k, pl.ds(o, L)], 6)
                    _ak(k, sl, 12,
                        st_i[1, k, pl.ds(o, L)], st_i[3, k, pl.ds(o, L)],
                        st_i[6, k, pl.ds(o, L)], 9)
                    return _u
                lax.fori_loop(0, 128 // L, gbody1, 0)
            ds_ = [pltpu.async_copy(obuf.at[ch],
                                    out_h.at[ch, pl.ds(r0, CR), :], sem)
                   for ch in range(6, 12)]
            for d in ds_:
                d.wait()
        return carry
    lax.fori_loop(0, S_STEPS, _gphase, 0)


def _sc_features(xs, ys, zs, vd, bi, x0, x1, y0, y1, z0, z1, ci):
    mesh = plsc.VectorSubcoreMesh(core_axis_name="c", subcore_axis_name="s",
                                  num_cores=NC, num_subcores=NS)
    f = pl.kernel(
        _sc_body,
        out_type=[jax.ShapeDtypeStruct((12, N_ROWS, 128), jnp.float32),
                  jax.ShapeDtypeStruct((N_ROWS, 128), jnp.float32)],
        mesh=mesh,
        scratch_types=(
            [pltpu.VMEM_SHARED((T0_ROWS,), jnp.float32)] * 4 +
            [pltpu.VMEM_SHARED((TK_ROWS,), jnp.float32)] * 12 +
            [
                pltpu.VMEM((7, CHUNK // 128, 128), jnp.int32),
                pltpu.VMEM((4, CHUNK // 128, 128), jnp.float32),
                pltpu.VMEM((CHUNK // 128, 128), jnp.int32),
                pltpu.VMEM((CHUNK // 128, 128), jnp.int32),
                pltpu.VMEM((CHUNK // 128, 128), jnp.int32),
                pltpu.VMEM((CHUNK // 128, 128), jnp.int32),
                pltpu.VMEM((16, CHUNK // 128, 128), jnp.float32),
                pltpu.VMEM((12, CHUNK // 128, 128), jnp.float32),
                pltpu.VMEM((CHUNK // 128, 128), jnp.float32),
                pltpu.VMEM((1024,), jnp.float32),
                pltpu.SemaphoreType.DMA,
            ]),
    )
    return f(xs, ys, zs, vd, bi, x0, x1, y0, y1, z0, z1, ci)


def _tc_body(coors_ref, p8_ref, sc_ref, recip_ref, w1e_ref, w1s_ref, b1_ref,
             w2_ref, b2_ref, out_ref):
    i = pl.program_id(0)

    @pl.when(i == 0)
    def _init():
        out_ref[...] = jnp.zeros_like(out_ref)

    h = jax.lax.dot_general(p8_ref[...], w1e_ref[...], (((1,), (0,)), ((), ())),
                            preferred_element_type=jnp.float32)
    h += jax.lax.dot_general(sc_ref[...], w1s_ref[...], (((0,), (0,)), ((), ())),
                             preferred_element_type=jnp.float32)
    h = jnp.maximum(h + b1_ref[...], 0.0)
    pf = jax.lax.dot_general(h, w2_ref[...], (((1,), (0,)), ((), ())),
                             preferred_element_type=jnp.float32)
    pf = (pf + b2_ref[...]) * recip_ref[...]

    pfb = pf.astype(jnp.bfloat16)
    idx = coors_ref[0, 0, :]
    lo = coors_ref[0, 0, 0]
    hi = coors_ref[0, 0, PT_BLK - 1]
    lo_al = (lo // VOX_CHUNK) * VOX_CHUNK
    nchunk = (hi - lo_al) // VOX_CHUNK + 1
    idx_row = idx.reshape(1, PT_BLK)

    def chunk_body(ch, carry):
        vbase = lo_al + ch * VOX_CHUNK
        rows = jax.lax.broadcasted_iota(jnp.int32, (VOX_CHUNK, PT_BLK), 0) + vbase
        oh = jnp.where(rows == idx_row, 1.0, 0.0).astype(jnp.bfloat16)
        contrib = jax.lax.dot_general(oh, pfb, (((1,), (0,)), ((), ())),
                                      preferred_element_type=jnp.float32)
        out_ref[pl.ds(vbase, VOX_CHUNK), :] += contrib
        return carry

    jax.lax.fori_loop(0, nchunk, chunk_body, 0)


def _mlp_segmean(coors3, p8, sc12, recip, W1e, W1s, b1e, W2, b2r):
    return pl.pallas_call(
        _tc_body,
        grid=(N_BLOCKS,),
        in_specs=[
            pl.BlockSpec((1, 1, PT_BLK), lambda i: (i, 0, 0)),
            pl.BlockSpec((PT_BLK, 8), lambda i: (i, 0)),
            pl.BlockSpec((12, PT_BLK), lambda i: (0, i)),
            pl.BlockSpec((PT_BLK, 1), lambda i: (i, 0)),
            pl.BlockSpec((8, 64), lambda i: (0, 0)),
            pl.BlockSpec((12, 64), lambda i: (0, 0)),
            pl.BlockSpec((1, 64), lambda i: (0, 0)),
            pl.BlockSpec((64, 64), lambda i: (0, 0)),
            pl.BlockSpec((1, 64), lambda i: (0, 0)),
        ],
        out_specs=pl.BlockSpec((NV_PAD, 64), lambda i: (0, 0)),
        out_shape=jax.ShapeDtypeStruct((NV_PAD, 64), jnp.float32),
        interpret=_INTERPRET,
    )(coors3, p8, sc12, recip, W1e, W1s, b1e, W2, b2r)


def _pad1(a, padval):
    pad = NP_PAD - N_POINTS
    return jnp.concatenate([a, jnp.full((pad,), padval, a.dtype)])


def kernel(points, batch_idx, full_coors, coors_inv, xidx0, xidx1, yidx0,
           yidx1, zidx0, zidx1, W1, bias1, W2, bias2):
    pad = NP_PAD - N_POINTS
    ar = jnp.arange(pad, dtype=jnp.int32)

    xs = _pad1(points[:, 0], 0.0).reshape(N_ROWS, 128)
    ys = _pad1(points[:, 1], 0.0).reshape(N_ROWS, 128)
    zs = _pad1(points[:, 2], 0.0).reshape(N_ROWS, 128)
    vd = jnp.concatenate([jnp.ones((N_POINTS,), jnp.float32),
                          jnp.zeros((pad,), jnp.float32)]).reshape(N_ROWS, 128)
    bi = _pad1(batch_idx, 0).reshape(N_ROWS, 128)
    x0p = _pad1(xidx0, 0).reshape(N_ROWS, 128)
    x1p = _pad1(xidx1, 0).reshape(N_ROWS, 128)
    y0p = _pad1(yidx0, 0).reshape(N_ROWS, 128)
    y1p = _pad1(yidx1, 0).reshape(N_ROWS, 128)
    z0p = _pad1(zidx0, 0).reshape(N_ROWS, 128)
    z1p = _pad1(zidx1, 0).reshape(N_ROWS, 128)
    cip = jnp.concatenate([coors_inv, N_VOXELS + (ar % (NV_PAD - N_VOXELS))])
    ci2 = cip.reshape(N_ROWS, 128)

    sc12, rec = _sc_features(xs, ys, zs, vd, bi, x0p, x1p, y0p, y1p, z0p, z1p,
                             ci2)
    sc12 = sc12.reshape(12, NP_PAD)
    recip = rec.reshape(NP_PAD, 1)

    iv = _INTERVALS
    mn = _MINS
    W1e = jnp.stack([
        W1[0] + W1[4] + W1[7],
        W1[1] + W1[5] + W1[8],
        W1[2] + W1[6] + W1[9],
        W1[3],
        -iv[0] * W1[7],
        -iv[1] * W1[8],
        -iv[2] * W1[9],
        jnp.zeros((64,), jnp.float32),
    ])
    b1e = (bias1 - mn[0] * W1[7] - mn[1] * W1[8] - mn[2] * W1[9]).reshape(1, 64)
    W1s = jnp.concatenate([-W1[4:7], W1[10:19]])

    p8 = jnp.concatenate([
        points, full_coors[:, 1:].astype(jnp.float32),
        jnp.zeros((N_POINTS, 1), jnp.float32)], axis=1)
    p8 = jnp.concatenate([p8, jnp.zeros((pad, 8), jnp.float32)], axis=0)
    coors3 = cip.reshape(N_BLOCKS, 1, PT_BLK)

    out = _mlp_segmean(coors3, p8, sc12, recip, W1e, W1s, b1e, W2,
                       bias2.reshape(1, 64))
    return out[:N_VOXELS]


# speedup vs baseline: 1.3647x; 1.0008x over previous
import jax
import jax.numpy as jnp
import numpy as np
from jax import lax
from jax.experimental import pallas as pl
from jax.experimental.pallas import tpu as pltpu
from jax.experimental.pallas import tpu_sc as plsc

N_VOXELS = 40000
N_POINTS = 160000
N_GRID = 32

PT_BLK = 2048
NP_PAD = 163840
N_ROWS = NP_PAD // 128
N_BLOCKS = NP_PAD // PT_BLK
NV_PAD = 40960
VOX_CHUNK = 512

NC, NS, L = 2, 16, 16
NW = NC * NS
T0_ROWS = NV_PAD
TK_ROWS = 131072
CHUNK = 256
PPW = NP_PAD // NW
PPT = NP_PAD // NS
G_STEPS = PPW // CHUNK
S_STEPS = PPT // CHUNK

_CRANGE = np.array([[-51.2, 51.2], [-51.2, 51.2], [-4.0, 2.4]], dtype=np.float32)
_SPATIAL = np.array([512.0, 512.0, 32.0], dtype=np.float32)
_INTERVALS = (_CRANGE[:, 1] - _CRANGE[:, 0]) / _SPATIAL
_MINS = _CRANGE[:, 0]


def _sc_body(xs_h, ys_h, zs_h, vd_h, bi_h, x0_h, x1_h, y0_h, y1_h, z0_h,
             z1_h, ci_h, out_h, rec_h, *refs):
    tabs = refs[:16]
    (st_i, st_f, idx0, idx1, idx2, idx3, chanv, obuf, rbuf,
     zbuf, sem) = refs[16:]
    c = lax.axis_index("c")
    s = lax.axis_index("s")
    w = c * NS + s

    def zf(i, _):
        zbuf[pl.ds(i * L, L)] = jnp.zeros((L,), jnp.float32)
        return _
    lax.fori_loop(0, 1024 // L, zf, 0)

    ZC = 1024
    t0_pt = T0_ROWS // NS
    for t in tabs[:4]:
        for k in range(t0_pt // ZC):
            pltpu.sync_copy(zbuf, t.at[pl.ds(s * t0_pt + k * ZC, ZC)])
        pltpu.sync_copy(zbuf.at[pl.ds(0, t0_pt % ZC)],
                        t.at[pl.ds(s * t0_pt + (t0_pt // ZC) * ZC,
                                   t0_pt % ZC)])
    tk_pt = TK_ROWS // NS
    for t in tabs[4:]:
        for k in range(tk_pt // ZC):
            pltpu.sync_copy(zbuf, t.at[pl.ds(s * tk_pt + k * ZC, ZC)])
    plsc.subcore_barrier()

    CR = CHUNK // 128

    def _stage(r0):
        ds_ = [
            pltpu.async_copy(bi_h.at[pl.ds(r0, CR), :], st_i.at[0], sem),
            pltpu.async_copy(x0_h.at[pl.ds(r0, CR), :], st_i.at[1], sem),
            pltpu.async_copy(x1_h.at[pl.ds(r0, CR), :], st_i.at[2], sem),
            pltpu.async_copy(y0_h.at[pl.ds(r0, CR), :], st_i.at[3], sem),
            pltpu.async_copy(y1_h.at[pl.ds(r0, CR), :], st_i.at[4], sem),
            pltpu.async_copy(z0_h.at[pl.ds(r0, CR), :], st_i.at[5], sem),
            pltpu.async_copy(z1_h.at[pl.ds(r0, CR), :], st_i.at[6], sem),
            pltpu.async_copy(ci_h.at[pl.ds(r0, CR), :], idx0, sem),
            pltpu.async_copy(xs_h.at[pl.ds(r0, CR), :], st_f.at[0], sem),
            pltpu.async_copy(ys_h.at[pl.ds(r0, CR), :], st_f.at[1], sem),
            pltpu.async_copy(zs_h.at[pl.ds(r0, CR), :], st_f.at[2], sem),
            pltpu.async_copy(vd_h.at[pl.ds(r0, CR), :], st_f.at[3], sem),
        ]
        for d in ds_:
            d.wait()

    def _idxcompute(k):
        def body(sl, _):
            o = sl * L
            bi = st_i[0, k, pl.ds(o, L)]
            x0 = st_i[1, k, pl.ds(o, L)]
            x1 = st_i[2, k, pl.ds(o, L)]
            y0 = st_i[3, k, pl.ds(o, L)]
            y1 = st_i[4, k, pl.ds(o, L)]
            z0 = st_i[5, k, pl.ds(o, L)]
            z1 = st_i[6, k, pl.ds(o, L)]
            idx1[k, pl.ds(o, L)] = ((bi * N_GRID + x1) * N_GRID + y0) * N_GRID + z0
            idx2[k, pl.ds(o, L)] = ((bi * N_GRID + x0) * N_GRID + y1) * N_GRID + z0
            idx3[k, pl.ds(o, L)] = ((bi * N_GRID + x0) * N_GRID + y0) * N_GRID + z1
            return _
        lax.fori_loop(0, 128 // L, body, 0)

    def _sphase(g, carry):
        _stage(s * (PPT // 128) + g * (CHUNK // 128))

        def _scat(groups):
            ds_ = []
            for k in range(CHUNK // 128):
                _idxcompute(k)
                for idx, base in groups:
                    ir = idx.at[k]
                    for ch in range(4):
                        ds_.append(pltpu.async_copy(st_f.at[ch, k],
                                                    tabs[base + ch].at[ir],
                                                    sem, add=True))
            for d in ds_:
                d.wait()

        @pl.when(c == 0)
        def _():
            _scat(((idx0, 0), (idx1, 4)))

        @pl.when(c == 1)
        def _():
            _scat(((idx2, 8), (idx3, 12)))
        return carry
    lax.fori_loop(0, S_STEPS, _sphase, 0)
    plsc.subcore_barrier()

    def _gphase(g, carry):
        r0 = s * (PPT // 128) + g * (CHUNK // 128)
        _stage(r0)

        def _gat(groups):
            ds_ = []
            for k in range(CHUNK // 128):
                _idxcompute(k)
                for idx, base in groups:
                    ir = idx.at[k]
                    for ch in range(4):
                        ds_.append(pltpu.async_copy(tabs[base + ch].at[ir],
                                                    chanv.at[base + ch, k],
                                                    sem))
            for d in ds_:
                d.wait()

        def _ak(k, sl, base, xi, yi, zi, cb):
            o = sl * L
            xs = st_f[0, k, pl.ds(o, L)]
            ys = st_f[1, k, pl.ds(o, L)]
            zs = st_f[2, k, pl.ds(o, L)]
            nk = jnp.maximum(chanv[base + 3, k, pl.ds(o, L)], 1.0)
            mx = chanv[base + 0, k, pl.ds(o, L)] / nk
            my = chanv[base + 1, k, pl.ds(o, L)] / nk
            mz = chanv[base + 2, k, pl.ds(o, L)] / nk
            dx = xs - (xi.astype(jnp.float32) * _INTERVALS[0] + _MINS[0])
            dy = ys - (yi.astype(jnp.float32) * _INTERVALS[1] + _MINS[1])
            dz = zs - (zi.astype(jnp.float32) * _INTERVALS[2] + _MINS[2])
            n2 = dx * dx + dy * dy + dz * dz
            bits = lax.bitcast_convert_type(n2, jnp.int32)
            y = lax.bitcast_convert_type(
                jnp.int32(0x5F3759DF) - lax.shift_right_logical(bits, 1),
                jnp.float32)
            for _i in range(3):
                y = y * (1.5 - 0.5 * n2 * y * y)
            obuf[cb + 0, k, pl.ds(o, L)] = (xs - mx) * y
            obuf[cb + 1, k, pl.ds(o, L)] = (ys - my) * y
            obuf[cb + 2, k, pl.ds(o, L)] = (zs - mz) * y

        @pl.when(c == 0)
        def _():
            _gat(((idx0, 0), (idx1, 4)))
            for k in range(CHUNK // 128):
                def gbody0(sl, _u):
                    o = sl * L
                    n0 = jnp.maximum(chanv[3, k, pl.ds(o, L)], 1.0)
                    obuf[0, k, pl.ds(o, L)] = chanv[0, k, pl.ds(o, L)] / n0
                    obuf[1, k, pl.ds(o, L)] = chanv[1, k, pl.ds(o, L)] / n0
                    obuf[2, k, pl.ds(o, L)] = chanv[2, k, pl.ds(o, L)] / n0
                    rbuf[k, pl.ds(o, L)] = 1.0 / n0
                    _ak(k, sl, 4,
                        st_i[2, k, pl.ds(o, L)], st_i[3, k, pl.ds(o, L)],
                        st_i[5, k, pl.ds(o, L)], 3)
                    return _u
                lax.fori_loop(0, 128 // L, gbody0, 0)
            ds_ = [pltpu.async_copy(obuf.at[ch],
                                    out_h.at[ch, pl.ds(r0, CR), :], sem)
                   for ch in range(6)]
            ds_.append(pltpu.async_copy(rbuf, rec_h.at[pl.ds(r0, CR), :], sem))
            for d in ds_:
                d.wait()

        @pl.when(c == 1)
        def _():
            _gat(((idx2, 8), (idx3, 12)))
            for k in range(CHUNK // 128):
                def gbody1(sl, _u):
                    o = sl * L
                    _ak(k, sl, 8,
                        st_i[1, k, pl.ds(o, L)], st_i[4, k, pl.ds(o, L)],
                        st_i[5, k, pl.ds(o, L)], 6)
                    _ak(k, sl, 12,
                        st_i[1, k, pl.ds(o, L)], st_i[3, k, pl.ds(o, L)],
                        st_i[6, k, pl.ds(o, L)], 9)
                    return _u
                lax.fori_loop(0, 128 // L, gbody1, 0)
            ds_ = [pltpu.async_copy(obuf.at[ch],
                                    out_h.at[ch, pl.ds(r0, CR), :], sem)
                   for ch in range(6, 12)]
            for d in ds_:
                d.wait()
        return carry
    lax.fori_loop(0, S_STEPS, _gphase, 0)


def _sc_features(xs, ys, zs, vd, bi, x0, x1, y0, y1, z0, z1, ci):
    mesh = plsc.VectorSubcoreMesh(core_axis_name="c", subcore_axis_name="s",
                                  num_cores=NC, num_subcores=NS)
    f = pl.kernel(
        _sc_body,
        out_type=[jax.ShapeDtypeStruct((12, N_ROWS, 128), jnp.float32),
                  jax.ShapeDtypeStruct((N_ROWS, 128), jnp.float32)],
        mesh=mesh,
        scratch_types=(
            [pltpu.VMEM_SHARED((T0_ROWS,), jnp.float32)] * 4 +
            [pltpu.VMEM_SHARED((TK_ROWS,), jnp.float32)] * 12 +
            [
                pltpu.VMEM((7, CHUNK // 128, 128), jnp.int32),
                pltpu.VMEM((4, CHUNK // 128, 128), jnp.float32),
                pltpu.VMEM((CHUNK // 128, 128), jnp.int32),
                pltpu.VMEM((CHUNK // 128, 128), jnp.int32),
                pltpu.VMEM((CHUNK // 128, 128), jnp.int32),
                pltpu.VMEM((CHUNK // 128, 128), jnp.int32),
                pltpu.VMEM((16, CHUNK // 128, 128), jnp.float32),
                pltpu.VMEM((12, CHUNK // 128, 128), jnp.float32),
                pltpu.VMEM((CHUNK // 128, 128), jnp.float32),
                pltpu.VMEM((1024,), jnp.float32),
                pltpu.SemaphoreType.DMA,
            ]),
    )
    return f(xs, ys, zs, vd, bi, x0, x1, y0, y1, z0, z1, ci)


def _tc_body(coors_ref, p8_ref, sc_ref, recip_ref, w1e_ref, w1s_ref, b1_ref,
             w2_ref, b2_ref, out_ref):
    i = pl.program_id(0)

    @pl.when(i == 0)
    def _init():
        out_ref[...] = jnp.zeros_like(out_ref)

    h = jax.lax.dot_general(p8_ref[...], w1e_ref[...], (((1,), (0,)), ((), ())),
                            preferred_element_type=jnp.float32)
    h += jax.lax.dot_general(sc_ref[...], w1s_ref[...], (((0,), (0,)), ((), ())),
                             preferred_element_type=jnp.float32)
    h = jnp.maximum(h + b1_ref[...], 0.0)
    pf = jax.lax.dot_general(h, w2_ref[...], (((1,), (0,)), ((), ())),
                             preferred_element_type=jnp.float32)
    pf = (pf + b2_ref[...]) * recip_ref[...]

    idx = coors_ref[0, 0, :]
    lo = coors_ref[0, 0, 0]
    hi = coors_ref[0, 0, PT_BLK - 1]
    lo_al = (lo // VOX_CHUNK) * VOX_CHUNK
    nchunk = (hi - lo_al) // VOX_CHUNK + 1
    idx_row = idx.reshape(1, PT_BLK)

    def chunk_body(ch, carry):
        vbase = lo_al + ch * VOX_CHUNK
        rows = jax.lax.broadcasted_iota(jnp.int32, (VOX_CHUNK, PT_BLK), 0) + vbase
        oh = jnp.where(rows == idx_row, 1.0, 0.0)
        contrib = jax.lax.dot_general(oh, pf, (((1,), (0,)), ((), ())),
                                      preferred_element_type=jnp.float32)
        out_ref[pl.ds(vbase, VOX_CHUNK), :] += contrib
        return carry

    jax.lax.fori_loop(0, nchunk, chunk_body, 0)


def _mlp_segmean(coors3, p8, sc12, recip, W1e, W1s, b1e, W2, b2r):
    return pl.pallas_call(
        _tc_body,
        grid=(N_BLOCKS,),
        in_specs=[
            pl.BlockSpec((1, 1, PT_BLK), lambda i: (i, 0, 0)),
            pl.BlockSpec((PT_BLK, 8), lambda i: (i, 0)),
            pl.BlockSpec((12, PT_BLK), lambda i: (0, i)),
            pl.BlockSpec((PT_BLK, 1), lambda i: (i, 0)),
            pl.BlockSpec((8, 64), lambda i: (0, 0)),
            pl.BlockSpec((12, 64), lambda i: (0, 0)),
            pl.BlockSpec((1, 64), lambda i: (0, 0)),
            pl.BlockSpec((64, 64), lambda i: (0, 0)),
            pl.BlockSpec((1, 64), lambda i: (0, 0)),
        ],
        out_specs=pl.BlockSpec((NV_PAD, 64), lambda i: (0, 0)),
        out_shape=jax.ShapeDtypeStruct((NV_PAD, 64), jnp.float32),
    )(coors3, p8, sc12, recip, W1e, W1s, b1e, W2, b2r)


def _pad1(a, padval):
    pad = NP_PAD - N_POINTS
    return jnp.concatenate([a, jnp.full((pad,), padval, a.dtype)])


def kernel(points, batch_idx, full_coors, coors_inv, xidx0, xidx1, yidx0,
           yidx1, zidx0, zidx1, W1, bias1, W2, bias2):
    pad = NP_PAD - N_POINTS
    ar = jnp.arange(pad, dtype=jnp.int32)

    xs = _pad1(points[:, 0], 0.0).reshape(N_ROWS, 128)
    ys = _pad1(points[:, 1], 0.0).reshape(N_ROWS, 128)
    zs = _pad1(points[:, 2], 0.0).reshape(N_ROWS, 128)
    vd = jnp.concatenate([jnp.ones((N_POINTS,), jnp.float32),
                          jnp.zeros((pad,), jnp.float32)]).reshape(N_ROWS, 128)
    bi = _pad1(batch_idx, 0).reshape(N_ROWS, 128)
    x0p = _pad1(xidx0, 0).reshape(N_ROWS, 128)
    x1p = _pad1(xidx1, 0).reshape(N_ROWS, 128)
    y0p = _pad1(yidx0, 0).reshape(N_ROWS, 128)
    y1p = _pad1(yidx1, 0).reshape(N_ROWS, 128)
    z0p = _pad1(zidx0, 0).reshape(N_ROWS, 128)
    z1p = _pad1(zidx1, 0).reshape(N_ROWS, 128)
    cip = jnp.concatenate([coors_inv, N_VOXELS + (ar % (NV_PAD - N_VOXELS))])
    ci2 = cip.reshape(N_ROWS, 128)

    sc12, rec = _sc_features(xs, ys, zs, vd, bi, x0p, x1p, y0p, y1p, z0p, z1p,
                             ci2)
    sc12 = sc12.reshape(12, NP_PAD)
    recip = rec.reshape(NP_PAD, 1)

    iv = _INTERVALS
    mn = _MINS
    W1e = jnp.stack([
        W1[0] + W1[4] + W1[7],
        W1[1] + W1[5] + W1[8],
        W1[2] + W1[6] + W1[9],
        W1[3],
        -iv[0] * W1[7],
        -iv[1] * W1[8],
        -iv[2] * W1[9],
        jnp.zeros((64,), jnp.float32),
    ])
    b1e = (bias1 - mn[0] * W1[7] - mn[1] * W1[8] - mn[2] * W1[9]).reshape(1, 64)
    W1s = jnp.concatenate([-W1[4:7], W1[10:19]])

    p8 = jnp.concatenate([
        points, full_coors[:, 1:].astype(jnp.float32),
        jnp.zeros((N_POINTS, 1), jnp.float32)], axis=1)
    p8 = jnp.concatenate([p8, jnp.zeros((pad, 8), jnp.float32)], axis=0)
    coors3 = cip.reshape(N_BLOCKS, 1, PT_BLK)

    out = _mlp_segmean(coors3, p8, sc12, recip, W1e, W1s, b1e, W2,
                       bias2.reshape(1, 64))
    return out[:N_VOXELS]
